# baseline (speedup 1.0000x reference)
import functools

import jax
import jax.numpy as jnp
from jax import lax
from jax.experimental import pallas as pl
from jax.experimental.pallas import tpu as pltpu
from jax.experimental.pallas import tpu_sc as plsc

_E = 8
_CH = 96
_IRS = 128
_HID = 128
_EMB = 64
_OH = 111
_RO = 16
_RT = 7
_W = 224
_WO = 128
_M3 = 3 * _CH
_MR = _RO * _CH
_L = 16


def _sc_routing_body(rv_hbm, w1_hbm, b1_hbm, w2_hbm, b2_hbm, emb_hbm, out_hbm,
                     rv_v, w1_v, b1_v, w2_v, b2_v, emb_v, h_v, r_v, idx_v):
    @pl.when(jnp.logical_and(lax.axis_index("c") == 0, lax.axis_index("s") == 0))
    def _():
        pltpu.sync_copy(rv_hbm, rv_v)
        pltpu.sync_copy(w1_hbm, w1_v)
        pltpu.sync_copy(b1_hbm, b1_v)
        pltpu.sync_copy(w2_hbm, w2_v)
        pltpu.sync_copy(b2_hbm, b2_v)
        pltpu.sync_copy(emb_hbm, emb_v)
        idx = jnp.zeros((_L,), jnp.int32)
        for b in range(2):
            for jc in range(_HID // _L):
                def l1(kc, acc):
                    rvc = rv_v[b, pl.ds(kc * _L, _L)]
                    for l in range(_L):
                        acc = acc + rvc[l] * w1_v[kc * _L + l, pl.ds(jc * _L, _L)]
                    return acc
                acc = lax.fori_loop(0, _IRS // _L, l1,
                                    jnp.zeros((_L,), jnp.float32))
                h_v[pl.ds(jc * _L, _L)] = jnp.maximum(
                    acc + b1_v[pl.ds(jc * _L, _L)], 0.0)
            for jc in range(_EMB // _L):
                def l2(kc, acc):
                    hc = h_v[pl.ds(kc * _L, _L)]
                    for l in range(_L):
                        acc = acc + hc[l] * w2_v[kc * _L + l, pl.ds(jc * _L, _L)]
                    return acc
                acc = lax.fori_loop(0, _HID // _L, l2,
                                    jnp.zeros((_L,), jnp.float32))
                r_v[pl.ds(jc * _L, _L)] = acc + b2_v[pl.ds(jc * _L, _L)]
            best_t2 = jnp.float32(-3.0e38)
            best_n = jnp.float32(1.0)
            bidx = jnp.int32(0)
            for e in range(_E):
                tv = jnp.zeros((_L,), jnp.float32)
                nv = jnp.zeros((_L,), jnp.float32)
                for jc in range(_EMB // _L):
                    ev = emb_v[e, pl.ds(jc * _L, _L)]
                    rv16 = r_v[pl.ds(jc * _L, _L)]
                    tv = tv + ev * rv16
                    nv = nv + ev * ev
                t = tv[0]
                n = nv[0]
                for l in range(1, _L):
                    t = t + tv[l]
                    n = n + nv[l]
                t2 = t * jnp.abs(t)
                n = jnp.maximum(n, jnp.float32(1e-16))
                better = t2 * best_n > best_t2 * n
                best_t2 = jnp.where(better, t2, best_t2)
                best_n = jnp.where(better, n, best_n)
                bidx = jnp.where(better, jnp.int32(e), bidx)
            lanes = lax.broadcasted_iota(jnp.int32, (_L,), 0)
            idx = jnp.where(lanes == b, bidx, idx)
        idx_v[...] = idx
        pltpu.sync_copy(idx_v, out_hbm)


_sc_routing = functools.partial(
    pl.kernel,
    out_type=jax.ShapeDtypeStruct((_L,), jnp.int32),
    mesh=plsc.VectorSubcoreMesh(core_axis_name="c", subcore_axis_name="s"),
    scratch_types=[
        pltpu.VMEM((2, _IRS), jnp.float32),
        pltpu.VMEM((_IRS, _HID), jnp.float32),
        pltpu.VMEM((_HID,), jnp.float32),
        pltpu.VMEM((_HID, _EMB), jnp.float32),
        pltpu.VMEM((_EMB,), jnp.float32),
        pltpu.VMEM((_E, _EMB), jnp.float32),
        pltpu.VMEM((_HID,), jnp.float32),
        pltpu.VMEM((_EMB,), jnp.float32),
        pltpu.VMEM((_L,), jnp.int32),
    ],
)(_sc_routing_body)


def _conv_body(idx_ref, xa_ref, xb_ref, w_ref, b_ref, o_ref, s_ref):
    del idx_ref
    wall = w_ref[0]
    for oh in range(_RO):
        lr = 2 * oh
        if lr + 3 <= 2 * _RO:
            r3 = xa_ref[0, lr:lr + 3].reshape(_M3, _W)
        else:
            r3 = jnp.concatenate(
                [xa_ref[0, lr:lr + 2].reshape(2 * _CH, _W),
                 xb_ref[0, 0:1].reshape(_CH, _W)], axis=0)
        t = jnp.dot(wall, r3, preferred_element_type=jnp.float32)
        tb = t.astype(jnp.bfloat16)
        for kw in range(3):
            s_ref[kw, oh * _CH:(oh + 1) * _CH, :] = tb[kw * _CH:(kw + 1) * _CH, :]
    acc = None
    for kw in range(3):
        ri = jax.lax.broadcasted_iota(jnp.int32, (_W, _WO), 0)
        ci = jax.lax.broadcasted_iota(jnp.int32, (_W, _WO), 1)
        sel = (ri == 2 * ci + kw).astype(jnp.bfloat16)
        d = jnp.dot(s_ref[kw], sel, preferred_element_type=jnp.float32)
        acc = d if acc is None else acc + d
    o_ref[0] = acc.reshape(_RO, _CH, _WO) + b_ref[...]


@jax.jit
def kernel(x, routing_vector, W1, b1, W2, b2, emb, convW, convB, task):
    B = x.shape[0]

    idx = _sc_routing(routing_vector, W1, b1, W2, b2, emb)[:B]

    xt = jnp.transpose(x, (0, 2, 1, 3)).astype(jnp.bfloat16)

    w3 = jnp.transpose(convW, (0, 4, 1, 3, 2)).reshape(_E, _M3, _M3).astype(jnp.bfloat16)
    bias = (convB + jnp.asarray(task, jnp.float32)).reshape(_E, _CH, 1)

    grid_spec = pltpu.PrefetchScalarGridSpec(
        num_scalar_prefetch=1,
        grid=(B, _RT),
        in_specs=[
            pl.BlockSpec((1, 2 * _RO, _CH, _W), lambda bb, r, sidx: (bb, r, 0, 0)),
            pl.BlockSpec((1, 2 * _RO, _CH, _W),
                         lambda bb, r, sidx: (bb, jnp.minimum(r + 1, _RT - 1), 0, 0)),
            pl.BlockSpec((1, _M3, _M3), lambda bb, r, sidx: (sidx[bb], 0, 0)),
            pl.BlockSpec((1, _CH, 1), lambda bb, r, sidx: (sidx[bb], 0, 0)),
        ],
        out_specs=pl.BlockSpec((1, _RO, _CH, _WO), lambda bb, r, sidx: (bb, r, 0, 0)),
        scratch_shapes=[pltpu.VMEM((3, _MR, _W), jnp.bfloat16)],
    )
    out = pl.pallas_call(
        _conv_body,
        grid_spec=grid_spec,
        out_shape=jax.ShapeDtypeStruct((B, _RT * _RO, _CH, _WO), jnp.float32),
    )(idx, xt, xt, w3, bias)

    return jnp.transpose(out, (0, 2, 1, 3))[:, :, :_OH, :_OH]
